# baseline (speedup 1.0000x reference)
import functools

import jax
import jax.numpy as jnp
from jax import lax
from jax.experimental import pallas as pl
from jax.experimental.pallas import tpu as pltpu
from jax.experimental.pallas import tpu_sc as plsc

B = 16384
NW = 32
BPW = B // NW
C = 128
NCH = BPW // C
L = 16


def _sc_body(u_idx, pi_idx, pc_idx, pb_idx, ni_idx, nc_idx, nb_idx,
             user_table, item_table, cat_table, brand_table,
             out_hbm,
             idx_v, u_rows, pi_rows, pc_rows, pb_rows,
             ni_rows, nc_rows, nb_rows, partial, sem):
    nc = jax.lax.axis_index("c")
    ns = jax.lax.axis_index("s")
    wid = ns * 2 + nc
    base = wid * BPW

    pltpu.sync_copy(u_idx.at[wid], idx_v.at[0])
    pltpu.sync_copy(pi_idx.at[wid], idx_v.at[1])
    pltpu.sync_copy(pc_idx.at[wid], idx_v.at[2])
    pltpu.sync_copy(pb_idx.at[wid], idx_v.at[3])
    pltpu.sync_copy(ni_idx.at[wid], idx_v.at[4])
    pltpu.sync_copy(nc_idx.at[wid], idx_v.at[5])
    pltpu.sync_copy(nb_idx.at[wid], idx_v.at[6])

    for c in range(NCH):
        cps = [
            pltpu.async_copy(user_table.at[idx_v.at[0, c]], u_rows, sem),
            pltpu.async_copy(item_table.at[idx_v.at[1, c]], pi_rows, sem),
            pltpu.async_copy(cat_table.at[idx_v.at[2, c]], pc_rows, sem),
            pltpu.async_copy(brand_table.at[idx_v.at[3, c]], pb_rows, sem),
            pltpu.async_copy(item_table.at[idx_v.at[4, c]], ni_rows, sem),
            pltpu.async_copy(cat_table.at[idx_v.at[5, c]], nc_rows, sem),
            pltpu.async_copy(brand_table.at[idx_v.at[6, c]], nb_rows, sem),
        ]
        for cp in cps:
            cp.wait()

        def row_body(r, _):
            acc = jnp.zeros((L,), jnp.float32)
            for k in range(4):
                d = pl.ds(k * L, L)
                acc += u_rows[r, d] * (ni_rows[r, d] - pi_rows[r, d])
            for k in range(2):
                d = pl.ds(k * L, L)
                du = pl.ds(64 + k * L, L)
                acc += u_rows[r, du] * (nc_rows[r, d] - pc_rows[r, d])
            for k in range(2):
                d = pl.ds(k * L, L)
                du = pl.ds(96 + k * L, L)
                acc += u_rows[r, du] * (nb_rows[r, d] - pb_rows[r, d])
            partial[r, :] = acc
            return 0

        lax.fori_loop(0, C, row_body, 0)
        pltpu.sync_copy(partial, out_hbm.at[pl.ds(base + c * C, C)])


@functools.partial(jax.jit, static_argnames=())
def _sc_partials(u_idx, pi_idx, pc_idx, pb_idx, ni_idx, nc_idx, nb_idx,
                 user_table, item_table, cat_table, brand_table):
    mesh = plsc.VectorSubcoreMesh(core_axis_name="c", subcore_axis_name="s")
    f = functools.partial(
        pl.kernel,
        mesh=mesh,
        compiler_params=pltpu.CompilerParams(use_tc_tiling_on_sc=False),
        out_type=jax.ShapeDtypeStruct((B, L), jnp.float32),
        scratch_types=[
            pltpu.VMEM((7, NCH, C), jnp.int32),
            pltpu.VMEM((C, 128), jnp.float32),
            pltpu.VMEM((C, 64), jnp.float32),
            pltpu.VMEM((C, 32), jnp.float32),
            pltpu.VMEM((C, 32), jnp.float32),
            pltpu.VMEM((C, 64), jnp.float32),
            pltpu.VMEM((C, 32), jnp.float32),
            pltpu.VMEM((C, 32), jnp.float32),
            pltpu.VMEM((C, L), jnp.float32),
            pltpu.SemaphoreType.DMA,
        ],
    )(_sc_body)
    return f(u_idx, pi_idx, pc_idx, pb_idx, ni_idx, nc_idx, nb_idx,
             user_table, item_table, cat_table, brand_table)


def _tc_loss_body(p_ref, o_ref):
    s = jnp.sum(p_ref[...], axis=1, keepdims=True)
    sp = jnp.maximum(s, 0.0) + jnp.log1p(jnp.exp(-jnp.abs(s)))
    o_ref[...] = (jnp.sum(sp) * (1.0 / B)).reshape(1, 1)


def _tc_loss(partials):
    out = pl.pallas_call(
        _tc_loss_body,
        out_shape=jax.ShapeDtypeStruct((1, 1), jnp.float32),
    )(partials)
    return out[0, 0]


def kernel(user, item, item_cat, item_brand, neg_item, neg_item_cat,
           neg_item_brand, user_table, item_table, cat_table, brand_table):
    def rs(x):
        return x.astype(jnp.int32).reshape(NW, NCH, C)

    partials = _sc_partials(rs(user), rs(item), rs(item_cat), rs(item_brand),
                            rs(neg_item), rs(neg_item_cat), rs(neg_item_brand),
                            user_table, item_table, cat_table, brand_table)
    return _tc_loss(partials)


# speedup vs baseline: 1.5252x; 1.5252x over previous
import functools

import jax
import jax.numpy as jnp
from jax import lax
from jax.experimental import pallas as pl
from jax.experimental.pallas import tpu as pltpu
from jax.experimental.pallas import tpu_sc as plsc

B = 16384
NW = 32
BPW = B // NW
C = 64
NCH = BPW // C
L = 16


def _sc_body(u_idx, pi_idx, pc_idx, pb_idx, ni_idx, nc_idx, nb_idx,
             user_table, item_table, cat_table, brand_table,
             out_hbm,
             idx_v, u_rows, pi_rows, pc_rows, pb_rows,
             ni_rows, nc_rows, nb_rows, partial,
             sem_u, sem_pi, sem_pc, sem_pb, sem_ni, sem_nc, sem_nb):
    nc = jax.lax.axis_index("c")
    ns = jax.lax.axis_index("s")
    wid = ns * 2 + nc
    base = wid * BPW

    pltpu.sync_copy(u_idx.at[wid], idx_v.at[0])
    pltpu.sync_copy(pi_idx.at[wid], idx_v.at[1])
    pltpu.sync_copy(pc_idx.at[wid], idx_v.at[2])
    pltpu.sync_copy(pb_idx.at[wid], idx_v.at[3])
    pltpu.sync_copy(ni_idx.at[wid], idx_v.at[4])
    pltpu.sync_copy(nc_idx.at[wid], idx_v.at[5])
    pltpu.sync_copy(nb_idx.at[wid], idx_v.at[6])

    for c in range(NCH):
        cp_u = pltpu.async_copy(user_table.at[idx_v.at[0, c]], u_rows, sem_u)

        def fire_group(g, _):
            b16 = g * L
            d16 = pl.ds(b16, L)
            vi_p = idx_v[1, c, d16]
            vc_p = idx_v[2, c, d16]
            vb_p = idx_v[3, c, d16]
            vi_n = idx_v[4, c, d16]
            vc_n = idx_v[5, c, d16]
            vb_n = idx_v[6, c, d16]
            for j in range(L):
                r = b16 + j
                pltpu.async_copy(item_table.at[vi_p[j]], pi_rows.at[r], sem_pi)
                pltpu.async_copy(cat_table.at[vc_p[j]], pc_rows.at[r], sem_pc)
                pltpu.async_copy(brand_table.at[vb_p[j]], pb_rows.at[r], sem_pb)
                pltpu.async_copy(item_table.at[vi_n[j]], ni_rows.at[r], sem_ni)
                pltpu.async_copy(cat_table.at[vc_n[j]], nc_rows.at[r], sem_nc)
                pltpu.async_copy(brand_table.at[vb_n[j]], nb_rows.at[r], sem_nb)
            return 0

        lax.fori_loop(0, C // L, fire_group, 0)

        pltpu.make_async_copy(item_table.at[pl.ds(0, C)], pi_rows, sem_pi).wait()
        pltpu.make_async_copy(cat_table.at[pl.ds(0, C)], pc_rows, sem_pc).wait()
        pltpu.make_async_copy(brand_table.at[pl.ds(0, C)], pb_rows, sem_pb).wait()
        pltpu.make_async_copy(item_table.at[pl.ds(0, C)], ni_rows, sem_ni).wait()
        pltpu.make_async_copy(cat_table.at[pl.ds(0, C)], nc_rows, sem_nc).wait()
        pltpu.make_async_copy(brand_table.at[pl.ds(0, C)], nb_rows, sem_nb).wait()
        cp_u.wait()

        def row_body(r, _):
            acc = jnp.zeros((L,), jnp.float32)
            for k in range(4):
                d = pl.ds(k * L, L)
                acc += u_rows[r, d] * (ni_rows[r, d] - pi_rows[r, d])
            for k in range(2):
                d = pl.ds(k * L, L)
                du = pl.ds(64 + k * L, L)
                acc += u_rows[r, du] * (nc_rows[r, d] - pc_rows[r, d])
            for k in range(2):
                d = pl.ds(k * L, L)
                du = pl.ds(96 + k * L, L)
                acc += u_rows[r, du] * (nb_rows[r, d] - pb_rows[r, d])
            partial[r, :] = acc
            return 0

        lax.fori_loop(0, C, row_body, 0)
        pltpu.sync_copy(partial, out_hbm.at[pl.ds(base + c * C, C)])


@functools.partial(jax.jit, static_argnames=())
def _sc_partials(u_idx, pi_idx, pc_idx, pb_idx, ni_idx, nc_idx, nb_idx,
                 user_table, item_table, cat_table, brand_table):
    mesh = plsc.VectorSubcoreMesh(core_axis_name="c", subcore_axis_name="s")
    f = functools.partial(
        pl.kernel,
        mesh=mesh,
        out_type=jax.ShapeDtypeStruct((B, L), jnp.float32),
        scratch_types=[
            pltpu.VMEM((7, NCH, C), jnp.int32),
            pltpu.VMEM((C, 128), jnp.float32),
            pltpu.VMEM((C, 64), jnp.float32),
            pltpu.VMEM((C, 32), jnp.float32),
            pltpu.VMEM((C, 32), jnp.float32),
            pltpu.VMEM((C, 64), jnp.float32),
            pltpu.VMEM((C, 32), jnp.float32),
            pltpu.VMEM((C, 32), jnp.float32),
            pltpu.VMEM((C, L), jnp.float32),
            pltpu.SemaphoreType.DMA,
            pltpu.SemaphoreType.DMA,
            pltpu.SemaphoreType.DMA,
            pltpu.SemaphoreType.DMA,
            pltpu.SemaphoreType.DMA,
            pltpu.SemaphoreType.DMA,
            pltpu.SemaphoreType.DMA,
        ],
    )(_sc_body)
    return f(u_idx, pi_idx, pc_idx, pb_idx, ni_idx, nc_idx, nb_idx,
             user_table, item_table, cat_table, brand_table)


def _tc_loss_body(p_ref, o_ref):
    s = jnp.sum(p_ref[...], axis=1, keepdims=True)
    sp = jnp.maximum(s, 0.0) + jnp.log1p(jnp.exp(-jnp.abs(s)))
    o_ref[...] = (jnp.sum(sp) * (1.0 / B)).reshape(1, 1)


def _tc_loss(partials):
    out = pl.pallas_call(
        _tc_loss_body,
        out_shape=jax.ShapeDtypeStruct((1, 1), jnp.float32),
    )(partials)
    return out[0, 0]


def kernel(user, item, item_cat, item_brand, neg_item, neg_item_cat,
           neg_item_brand, user_table, item_table, cat_table, brand_table):
    def rs(x):
        return x.astype(jnp.int32).reshape(NW, NCH, C)

    partials = _sc_partials(rs(user), rs(item), rs(item_cat), rs(item_brand),
                            rs(neg_item), rs(neg_item_cat), rs(neg_item_brand),
                            user_table, item_table, cat_table, brand_table)
    return _tc_loss(partials)
